# baseline (speedup 1.0000x reference)
import functools

import jax
import jax.numpy as jnp
from jax import lax
from jax.experimental import pallas as pl
from jax.experimental.pallas import tpu as pltpu
from jax.experimental.pallas import tpu_sc as plsc

EMB = 32
NB = 4096
NT = 200
B_TOTAL = NB * NT
NUM_WORKERS = 32
ROWS_PER_WORKER = NB // NUM_WORKERS
PER_WORKER = ROWS_PER_WORKER * NT
CHUNK_ROWS = 8
CHUNK = CHUNK_ROWS * NT
N_STEPS = ROWS_PER_WORKER // CHUNK_ROWS


@functools.partial(
    pl.kernel,
    out_type=jax.ShapeDtypeStruct((NB, NT, EMB), jnp.float32),
    mesh=plsc.VectorSubcoreMesh(core_axis_name="c", subcore_axis_name="s"),
    scratch_types=[
        pltpu.VMEM((PER_WORKER,), jnp.int32),
        pltpu.VMEM((CHUNK, EMB), jnp.float32),
        pltpu.VMEM((CHUNK, EMB), jnp.float32),
        pltpu.SemaphoreType.DMA,
        pltpu.SemaphoreType.DMA,
        pltpu.SemaphoreType.DMA,
        pltpu.SemaphoreType.DMA,
    ],
    compiler_params=pltpu.CompilerParams(use_tc_tiling_on_sc=False),
)
def _gather_all(x_hbm, table_hbm, out_hbm, idx_v, rows0, rows1, g0, g1, o0, o1):
    wid = lax.axis_index("s") * 2 + lax.axis_index("c")
    base = wid * PER_WORKER
    row_base = wid * ROWS_PER_WORKER

    rows = (rows0, rows1)
    gsem = (g0, g1)
    osem = (o0, o1)

    pltpu.sync_copy(x_hbm.at[pl.ds(base, PER_WORKER)], idx_v)

    def start_gather(i, b):
        return pltpu.async_copy(
            table_hbm.at[idx_v.at[pl.ds(i * CHUNK, CHUNK)]], rows[b], gsem[b])

    def start_store(i, b):
        d = None
        for r in range(CHUNK_ROWS):
            d = pltpu.async_copy(
                rows[b].at[pl.ds(r * NT, NT)],
                out_hbm.at[row_base + i * CHUNK_ROWS + r],
                osem[b])
        return d

    def wait_store(b):
        for r in range(CHUNK_ROWS):
            pltpu.make_async_copy(
                rows[b].at[pl.ds(r * NT, NT)],
                out_hbm.at[row_base + r],
                osem[b]).wait()

    gd = [None, None]
    started = [False, False]
    gd[0] = start_gather(0, 0)
    for i in range(N_STEPS):
        b = i & 1
        nb = b ^ 1
        if i + 1 < N_STEPS:
            if started[nb]:
                wait_store(nb)
                started[nb] = False
            gd[nb] = start_gather(i + 1, nb)
        gd[b].wait()
        start_store(i, b)
        started[b] = True
    for b in (0, 1):
        if started[b]:
            wait_store(b)


def kernel(X, table):
    xf = X.reshape(-1)
    return _gather_all(xf, table)
